# baseline (speedup 1.0000x reference)
import functools

import jax
import jax.numpy as jnp
from jax import lax
from jax.experimental import pallas as pl
from jax.experimental.pallas import tpu as pltpu
from jax.experimental.pallas import tpu_sc as plsc

_N = 10000
_E = 320000
_HID = 128
_R = 2
_P = 0.5

_NC = 2
_NS = 16
_NW = _NC * _NS
_EPW = _E // _NW
_K = 64
_EPWP = 10240
_NCH = _EPWP // _K
_NBUF = 5
_ACCR = _N + 8
_RPT = _N // _NS


def _sc_agg_body(hf, eidx_hbm, out, idxb, ring, acc,
                 i0, i1, i2, i3, i4, g0, g1, g2, g3, g4,
                 s0, s1, s2, s3, s4):
    c = lax.axis_index("c")
    s = lax.axis_index("s")
    wid = c * _NS + s
    isems = (i0, i1, i2, i3, i4)
    gsems = (g0, g1, g2, g3, g4)
    ssems = (s0, s1, s2, s3, s4)

    zero = jnp.zeros((16,), jnp.float32)

    def zbody(i, carry):
        for lane0 in range(_HID // 16):
            ring[0, i, pl.ds(lane0 * 16, 16)] = zero
        return carry

    lax.fori_loop(0, _K, zbody, 0)
    for z in range(_RPT // _K):
        pltpu.sync_copy(ring.at[0], acc.at[pl.ds(s * _RPT + z * _K, _K)])
    pltpu.sync_copy(ring.at[0, pl.ds(0, _RPT % _K)],
                    acc.at[pl.ds(s * _RPT + (_RPT // _K) * _K, _RPT % _K)])
    plsc.subcore_barrier()

    def i_start(b, j):
        pltpu.async_copy(eidx_hbm.at[wid, j], idxb.at[b], isems[b])

    def i_wait(b):
        pltpu.make_async_copy(eidx_hbm.at[wid, 0], idxb.at[b],
                              isems[b]).wait()

    def g_start(b):
        pltpu.async_copy(hf.at[idxb.at[b, 0]], ring.at[b], gsems[b])

    def g_wait(b):
        pltpu.make_async_copy(hf.at[idxb.at[0, 0]], ring.at[b],
                              gsems[b]).wait()

    def s_start(b):
        pltpu.async_copy(ring.at[b], acc.at[idxb.at[b, 1]], ssems[b],
                         add=True)

    def s_wait(b):
        pltpu.make_async_copy(ring.at[b], acc.at[idxb.at[0, 1]],
                              ssems[b]).wait()

    for b in range(_NBUF):
        i_start(b, b)
    for b in range(_NBUF):
        i_wait(b)
        g_start(b)

    def body(i, carry):
        q = i * _NBUF
        for b in range(_NBUF):
            g_wait(b)
            s_start(b)
        for b in range(_NBUF):
            s_wait(b)
            i_start(b, q + _NBUF + b)
        for b in range(_NBUF):
            i_wait(b)
            g_start(b)
        return carry

    lax.fori_loop(0, _NCH // _NBUF - 1, body, 0)
    for b in range(_NBUF):
        g_wait(b)
        s_start(b)
    for b in range(_NBUF):
        s_wait(b)

    plsc.subcore_barrier()
    pltpu.sync_copy(acc.at[pl.ds(s * 624, 624)],
                    out.at[c, pl.ds(s * 624, 624)])

    @pl.when(s == 0)
    def _flush_tail():
        pltpu.sync_copy(acc.at[pl.ds(_NS * 624, _N - _NS * 624)],
                        out.at[c, pl.ds(_NS * 624, _N - _NS * 624)])


@functools.cache
def _get_sc_aggregate():
  return pl.kernel(
    _sc_agg_body,
    out_type=jax.ShapeDtypeStruct((_NC, _N, _HID), jnp.float32),
    mesh=plsc.VectorSubcoreMesh(core_axis_name="c", subcore_axis_name="s"),
    scratch_types=[
        pltpu.VMEM((_NBUF, 2, _K), jnp.int32),
        pltpu.VMEM((_NBUF, _K, _HID), jnp.float32),
        pltpu.VMEM_SHARED((_ACCR, _HID), jnp.float32),
    ] + [pltpu.SemaphoreType.DMA] * (3 * _NBUF),
  )


def _proj_body(x, w0t, b0, m0, m1, hf):
    xp = jnp.dot(x[...], w0t[...], preferred_element_type=jnp.float32) + b0[...]
    hf[pl.ds(0, _N), :] = xp * m0[...]
    hf[pl.ds(_N, _N), :] = xp * m1[...]


_proj = pl.pallas_call(
    _proj_body,
    out_shape=jax.ShapeDtypeStruct((_R * _N, _HID), jnp.float32),
)


def _layer_body(hf, a, w1t, b1, g, bt, w2t, b2, eps, o):
    e = 1.0 + eps[...]
    h0 = hf[pl.ds(0, _N), :]
    h1 = hf[pl.ds(_N, _N), :]
    pre0 = h0 * e + (a[0] + a[1])
    pre1 = h1 * e
    z0 = jnp.dot(pre0, w1t[...], preferred_element_type=jnp.float32) + b1[...]
    z1 = jnp.dot(pre1, w1t[...], preferred_element_type=jnp.float32) + b1[...]
    m = (jnp.sum(z0, 0, keepdims=True) + jnp.sum(z1, 0, keepdims=True)) * (
        0.5 / _N)
    d0 = z0 - m
    d1 = z1 - m
    v = (jnp.sum(d0 * d0, 0, keepdims=True)
         + jnp.sum(d1 * d1, 0, keepdims=True)) * (0.5 / _N)
    scale = lax.rsqrt(v + 1e-5) * g[...]
    r0 = jnp.maximum(d0 * scale + bt[...], 0.0)
    r1 = jnp.maximum(d1 * scale + bt[...], 0.0)
    o[pl.ds(0, _N), :] = jnp.maximum(
        jnp.dot(r0, w2t[...], preferred_element_type=jnp.float32) + b2[...], 0.0)
    o[pl.ds(_N, _N), :] = jnp.maximum(
        jnp.dot(r1, w2t[...], preferred_element_type=jnp.float32) + b2[...], 0.0)


_layer = pl.pallas_call(
    _layer_body,
    out_shape=jax.ShapeDtypeStruct((_R * _N, _HID), jnp.float32),
)


def _fin_body(hf, wft, bf, o):
    hm = (hf[pl.ds(0, _N), :] + hf[pl.ds(_N, _N), :]) * 0.5
    o[...] = jnp.dot(hm, wft[...], preferred_element_type=jnp.float32) + bf[...]


_fin = pl.pallas_call(
    _fin_body,
    out_shape=jax.ShapeDtypeStruct((_N, 40), jnp.float32),
)


def kernel(x_author, edge_index, W0, b0, W1_0, b1_0, g_0, bt_0, W2_0, b2_0,
           eps_0, W1_1, b1_1, g_1, bt_1, W2_1, b2_1, eps_1, Wf, bf):
    mask = jax.random.bernoulli(
        jax.random.key(42), 1.0 - _P, (_R, _N)).astype(jnp.float32)
    mask = mask * (1.0 / (1.0 - _P))
    m0 = mask[0][:, None]
    m1 = mask[1][:, None]
    npad = _NW * (_EPWP - _EPW)
    srcp = jnp.concatenate(
        [edge_index[0].reshape(_NW, _EPW),
         (jnp.arange(npad, dtype=jnp.int32) % _N).reshape(_NW, -1)], axis=1)
    dstp = jnp.concatenate(
        [edge_index[1].reshape(_NW, _EPW),
         (_N + jnp.arange(npad, dtype=jnp.int32) % 8).reshape(_NW, -1)],
        axis=1)
    eidx = jnp.stack([srcp.reshape(_NW, _NCH, _K),
                      dstp.reshape(_NW, _NCH, _K)], axis=2)

    hf = _proj(x_author, W0.T, b0.reshape(1, -1), m0, m1)

    w1ts = jnp.stack([W1_0.T, W1_1.T])
    b1s = jnp.stack([b1_0, b1_1]).reshape(_R, 1, _HID)
    gs = jnp.stack([g_0, g_1]).reshape(_R, 1, _HID)
    bts = jnp.stack([bt_0, bt_1]).reshape(_R, 1, _HID)
    w2ts = jnp.stack([W2_0.T, W2_1.T])
    b2s = jnp.stack([b2_0, b2_1]).reshape(_R, 1, _HID)
    epss = jnp.stack([eps_0, eps_1]).reshape(_R, 1, 1)

    nlayers = 2 + lax.optimization_barrier(jnp.zeros((), jnp.int32))
    stacked = (w1ts, b1s, gs, bts, w2ts, b2s, epss)

    def cond(carry):
        return carry[0] < nlayers

    def step(carry):
        i, hfc = carry
        w1t, b1r, gr, btr, w2t, b2r, epsr = (
            lax.dynamic_index_in_dim(w, i, keepdims=False) for w in stacked)
        agg = _get_sc_aggregate()(hfc, eidx)
        hfc = _layer(hfc, agg, w1t, b1r, gr, btr, w2t, b2r, epsr)
        return (i + 1, hfc)

    _, hf = lax.while_loop(cond, step, (jnp.int32(0), hf))

    return _fin(hf, Wf.T, bf.reshape(1, -1))


# speedup vs baseline: 1.0187x; 1.0187x over previous
import functools

import jax
import jax.numpy as jnp
from jax import lax
from jax.experimental import pallas as pl
from jax.experimental.pallas import tpu as pltpu
from jax.experimental.pallas import tpu_sc as plsc

_N = 10000
_E = 320000
_HID = 128
_R = 2
_P = 0.5

_NC = 2
_NS = 16
_NW = _NC * _NS
_EPW = _E // _NW
_K = 80
_NCH = _EPW // _K
_NBUF = 4
_RPT = _N // _NS


def _sc_agg_body(hf, eidx_hbm, out, idxb, ring, acc,
                 i0, i1, i2, i3, g0, g1, g2, g3, s0, s1, s2, s3):
    c = lax.axis_index("c")
    s = lax.axis_index("s")
    wid = c * _NS + s
    isems = (i0, i1, i2, i3)
    gsems = (g0, g1, g2, g3)
    ssems = (s0, s1, s2, s3)

    zero = jnp.zeros((16,), jnp.float32)

    def zbody(i, carry):
        for lane0 in range(_HID // 16):
            ring[0, i, pl.ds(lane0 * 16, 16)] = zero
        return carry

    lax.fori_loop(0, _K, zbody, 0)
    for z in range(_RPT // _K):
        pltpu.sync_copy(ring.at[0], acc.at[pl.ds(s * _RPT + z * _K, _K)])
    pltpu.sync_copy(ring.at[0, pl.ds(0, _RPT % _K)],
                    acc.at[pl.ds(s * _RPT + (_RPT // _K) * _K, _RPT % _K)])
    plsc.subcore_barrier()

    def i_start(b, j):
        pltpu.async_copy(eidx_hbm.at[wid, j], idxb.at[b], isems[b])

    def i_wait(b):
        pltpu.make_async_copy(eidx_hbm.at[wid, 0], idxb.at[b],
                              isems[b]).wait()

    def g_start(b):
        pltpu.async_copy(hf.at[idxb.at[b, 0]], ring.at[b], gsems[b])

    def g_wait(b):
        pltpu.make_async_copy(hf.at[idxb.at[0, 0]], ring.at[b],
                              gsems[b]).wait()

    def s_start(b):
        pltpu.async_copy(ring.at[b], acc.at[idxb.at[b, 1]], ssems[b],
                         add=True)

    def s_wait(b):
        pltpu.make_async_copy(ring.at[b], acc.at[idxb.at[0, 1]],
                              ssems[b]).wait()

    for b in range(_NBUF):
        i_start(b, b)
    for b in range(_NBUF):
        i_wait(b)
        g_start(b)

    def body(i, carry):
        q = i * _NBUF
        for b in range(_NBUF):
            g_wait(b)
            s_start(b)
        for b in range(_NBUF):
            s_wait(b)
            i_start(b, q + _NBUF + b)
        for b in range(_NBUF):
            i_wait(b)
            g_start(b)
        return carry

    lax.fori_loop(0, _NCH // _NBUF - 1, body, 0)
    for b in range(_NBUF):
        g_wait(b)
        s_start(b)
    s_wait(0)
    i_start(0, _NCH - 1)
    i_wait(0)
    g_start(0)
    g_wait(0)
    s_start(0)
    s_wait(0)
    for b in range(1, _NBUF):
        s_wait(b)

    plsc.subcore_barrier()
    pltpu.sync_copy(acc.at[pl.ds(s * 624, 624)],
                    out.at[c, pl.ds(s * 624, 624)])

    @pl.when(s == 0)
    def _flush_tail():
        pltpu.sync_copy(acc.at[pl.ds(_NS * 624, _N - _NS * 624)],
                        out.at[c, pl.ds(_NS * 624, _N - _NS * 624)])


@functools.cache
def _get_sc_aggregate():
  return pl.kernel(
    _sc_agg_body,
    out_type=jax.ShapeDtypeStruct((_NC, _N, _HID), jnp.float32),
    mesh=plsc.VectorSubcoreMesh(core_axis_name="c", subcore_axis_name="s"),
    scratch_types=[
        pltpu.VMEM((_NBUF, 2, _K), jnp.int32),
        pltpu.VMEM((_NBUF, _K, _HID), jnp.float32),
        pltpu.VMEM_SHARED((_N, _HID), jnp.float32),
        pltpu.SemaphoreType.DMA,
        pltpu.SemaphoreType.DMA,
        pltpu.SemaphoreType.DMA,
        pltpu.SemaphoreType.DMA,
        pltpu.SemaphoreType.DMA,
        pltpu.SemaphoreType.DMA,
        pltpu.SemaphoreType.DMA,
        pltpu.SemaphoreType.DMA,
        pltpu.SemaphoreType.DMA,
        pltpu.SemaphoreType.DMA,
        pltpu.SemaphoreType.DMA,
        pltpu.SemaphoreType.DMA,
    ],
  )


def _proj_body(x, w0t, b0, m0, m1, hf):
    xp = jnp.dot(x[...], w0t[...], preferred_element_type=jnp.float32) + b0[...]
    hf[pl.ds(0, _N), :] = xp * m0[...]
    hf[pl.ds(_N, _N), :] = xp * m1[...]


_proj = pl.pallas_call(
    _proj_body,
    out_shape=jax.ShapeDtypeStruct((_R * _N, _HID), jnp.float32),
)


def _layer_pre_body(hf, w1t, b1, eps, z1o):
    e = 1.0 + eps[...]
    h1 = hf[pl.ds(_N, _N), :]
    z1o[...] = jnp.dot(h1 * e, w1t[...],
                       preferred_element_type=jnp.float32) + b1[...]


_layer_pre = pl.pallas_call(
    _layer_pre_body,
    out_shape=jax.ShapeDtypeStruct((_N, _HID), jnp.float32),
)


def _layer_body(hf, a, z1in, w1t, b1, g, bt, w2t, b2, eps, o):
    e = 1.0 + eps[...]
    h0 = hf[pl.ds(0, _N), :]
    pre0 = h0 * e + (a[0] + a[1])
    z0 = jnp.dot(pre0, w1t[...], preferred_element_type=jnp.float32) + b1[...]
    z1 = z1in[...]
    m = (jnp.sum(z0, 0, keepdims=True) + jnp.sum(z1, 0, keepdims=True)) * (
        0.5 / _N)
    d0 = z0 - m
    d1 = z1 - m
    v = (jnp.sum(d0 * d0, 0, keepdims=True)
         + jnp.sum(d1 * d1, 0, keepdims=True)) * (0.5 / _N)
    scale = lax.rsqrt(v + 1e-5) * g[...]
    r0 = jnp.maximum(d0 * scale + bt[...], 0.0)
    r1 = jnp.maximum(d1 * scale + bt[...], 0.0)
    o[pl.ds(0, _N), :] = jnp.maximum(
        jnp.dot(r0, w2t[...], preferred_element_type=jnp.float32) + b2[...], 0.0)
    o[pl.ds(_N, _N), :] = jnp.maximum(
        jnp.dot(r1, w2t[...], preferred_element_type=jnp.float32) + b2[...], 0.0)


_layer = pl.pallas_call(
    _layer_body,
    out_shape=jax.ShapeDtypeStruct((_R * _N, _HID), jnp.float32),
)


def _fin_body(hf, wft, bf, o):
    hm = (hf[pl.ds(0, _N), :] + hf[pl.ds(_N, _N), :]) * 0.5
    o[...] = jnp.dot(hm, wft[...], preferred_element_type=jnp.float32) + bf[...]


_fin = pl.pallas_call(
    _fin_body,
    out_shape=jax.ShapeDtypeStruct((_N, 40), jnp.float32),
)


def kernel(x_author, edge_index, W0, b0, W1_0, b1_0, g_0, bt_0, W2_0, b2_0,
           eps_0, W1_1, b1_1, g_1, bt_1, W2_1, b2_1, eps_1, Wf, bf):
    mask = jax.random.bernoulli(
        jax.random.key(42), 1.0 - _P, (_R, _N)).astype(jnp.float32)
    mask = mask * (1.0 / (1.0 - _P))
    m0 = mask[0][:, None]
    m1 = mask[1][:, None]
    eidx = jnp.stack([edge_index[0].reshape(_NW, _NCH, _K),
                      edge_index[1].reshape(_NW, _NCH, _K)], axis=2)

    hf = _proj(x_author, W0.T, b0.reshape(1, -1), m0, m1)

    w1ts = jnp.stack([W1_0.T, W1_1.T])
    b1s = jnp.stack([b1_0, b1_1]).reshape(_R, 1, _HID)
    gs = jnp.stack([g_0, g_1]).reshape(_R, 1, _HID)
    bts = jnp.stack([bt_0, bt_1]).reshape(_R, 1, _HID)
    w2ts = jnp.stack([W2_0.T, W2_1.T])
    b2s = jnp.stack([b2_0, b2_1]).reshape(_R, 1, _HID)
    epss = jnp.stack([eps_0, eps_1]).reshape(_R, 1, 1)

    nlayers = 2 + lax.optimization_barrier(jnp.zeros((), jnp.int32))
    stacked = (w1ts, b1s, gs, bts, w2ts, b2s, epss)

    def cond(carry):
        return carry[0] < nlayers

    def step(carry):
        i, hfc = carry
        w1t, b1r, gr, btr, w2t, b2r, epsr = (
            lax.dynamic_index_in_dim(w, i, keepdims=False) for w in stacked)
        agg = _get_sc_aggregate()(hfc, eidx)
        z1 = _layer_pre(hfc, w1t, b1r, epsr)
        hfc = _layer(hfc, agg, z1, w1t, b1r, gr, btr, w2t, b2r, epsr)
        return (i + 1, hfc)

    _, hf = lax.while_loop(cond, step, (jnp.int32(0), hf))

    return _fin(hf, Wf.T, bf.reshape(1, -1))


# speedup vs baseline: 1.0281x; 1.0092x over previous
import functools

import jax
import jax.numpy as jnp
from jax import lax
from jax.experimental import pallas as pl
from jax.experimental.pallas import tpu as pltpu
from jax.experimental.pallas import tpu_sc as plsc

_N = 10000
_E = 320000
_HID = 128
_R = 2
_P = 0.5

_NC = 2
_NS = 16
_NW = _NC * _NS
_EPW = _E // _NW
_K = 80
_NCH = _EPW // _K
_NBUF = 4
_RPT = _N // _NS


def _sc_agg_body(hf, eidx_hbm, out, idxb, ring, acc,
                 i0, i1, i2, i3, g0, g1, g2, g3, s0, s1, s2, s3):
    c = lax.axis_index("c")
    s = lax.axis_index("s")
    wid = c * _NS + s
    isems = (i0, i1, i2, i3)
    gsems = (g0, g1, g2, g3)
    ssems = (s0, s1, s2, s3)

    zero = jnp.zeros((16,), jnp.float32)

    def zbody(i, carry):
        for lane0 in range(_HID // 16):
            ring[0, i, pl.ds(lane0 * 16, 16)] = zero
        return carry

    lax.fori_loop(0, _K, zbody, 0)
    for z in range(_RPT // _K):
        pltpu.sync_copy(ring.at[0], acc.at[pl.ds(s * _RPT + z * _K, _K)])
    pltpu.sync_copy(ring.at[0, pl.ds(0, _RPT % _K)],
                    acc.at[pl.ds(s * _RPT + (_RPT // _K) * _K, _RPT % _K)])
    plsc.subcore_barrier()

    def i_start(b, j):
        pltpu.async_copy(eidx_hbm.at[wid, j], idxb.at[b], isems[b])

    def i_wait(b):
        pltpu.make_async_copy(eidx_hbm.at[wid, 0], idxb.at[b],
                              isems[b]).wait()

    def g_start(b):
        pltpu.async_copy(hf.at[idxb.at[b, 0]], ring.at[b], gsems[b])

    def g_wait(b):
        pltpu.make_async_copy(hf.at[idxb.at[0, 0]], ring.at[b],
                              gsems[b]).wait()

    def s_start(b):
        pltpu.async_copy(ring.at[b], acc.at[idxb.at[b, 1]], ssems[b],
                         add=True)

    def s_wait(b):
        pltpu.make_async_copy(ring.at[b], acc.at[idxb.at[0, 1]],
                              ssems[b]).wait()

    for b in range(_NBUF):
        i_start(b, b)
    for b in range(_NBUF):
        i_wait(b)
        g_start(b)

    def body(i, carry):
        q = i * _NBUF
        for b in range(_NBUF):
            g_wait(b)
            s_start(b)
        for b in range(_NBUF):
            s_wait(b)
            i_start(b, q + _NBUF + b)
        for b in range(_NBUF):
            i_wait(b)
            g_start(b)
        return carry

    lax.fori_loop(0, _NCH // _NBUF - 1, body, 0)
    for b in range(_NBUF):
        g_wait(b)
        s_start(b)
    s_wait(0)
    i_start(0, _NCH - 1)
    i_wait(0)
    g_start(0)
    g_wait(0)
    s_start(0)
    s_wait(0)
    for b in range(1, _NBUF):
        s_wait(b)

    plsc.subcore_barrier()
    pltpu.sync_copy(acc.at[pl.ds(s * 624, 624)],
                    out.at[c, pl.ds(s * 624, 624)])

    @pl.when(s == 0)
    def _flush_tail():
        pltpu.sync_copy(acc.at[pl.ds(_NS * 624, _N - _NS * 624)],
                        out.at[c, pl.ds(_NS * 624, _N - _NS * 624)])


@functools.cache
def _get_sc_aggregate():
  return pl.kernel(
    _sc_agg_body,
    out_type=jax.ShapeDtypeStruct((_NC, _N, _HID), jnp.float32),
    mesh=plsc.VectorSubcoreMesh(core_axis_name="c", subcore_axis_name="s"),
    scratch_types=[
        pltpu.VMEM((_NBUF, 2, _K), jnp.int32),
        pltpu.VMEM((_NBUF, _K, _HID), jnp.float32),
        pltpu.VMEM_SHARED((_N, _HID), jnp.float32),
        pltpu.SemaphoreType.DMA,
        pltpu.SemaphoreType.DMA,
        pltpu.SemaphoreType.DMA,
        pltpu.SemaphoreType.DMA,
        pltpu.SemaphoreType.DMA,
        pltpu.SemaphoreType.DMA,
        pltpu.SemaphoreType.DMA,
        pltpu.SemaphoreType.DMA,
        pltpu.SemaphoreType.DMA,
        pltpu.SemaphoreType.DMA,
        pltpu.SemaphoreType.DMA,
        pltpu.SemaphoreType.DMA,
    ],
  )


def _proj_body(x, w0t, b0, m0, m1, hf):
    xp = jnp.dot(x[...], w0t[...], preferred_element_type=jnp.float32) + b0[...]
    hf[pl.ds(0, _N), :] = xp * m0[...]
    hf[pl.ds(_N, _N), :] = xp * m1[...]


_proj = pl.pallas_call(
    _proj_body,
    out_shape=jax.ShapeDtypeStruct((_R * _N, _HID), jnp.float32),
)


def _layer_body(hf, a, w1t, b1, g, bt, w2t, b2, eps, o):
    e = 1.0 + eps[...]
    h0 = hf[pl.ds(0, _N), :]
    h1 = hf[pl.ds(_N, _N), :]
    pre0 = h0 * e + (a[0] + a[1])
    pre1 = h1 * e
    z0 = jnp.dot(pre0, w1t[...], preferred_element_type=jnp.float32) + b1[...]
    z1 = jnp.dot(pre1, w1t[...], preferred_element_type=jnp.float32) + b1[...]
    m = (jnp.sum(z0, 0, keepdims=True) + jnp.sum(z1, 0, keepdims=True)) * (
        0.5 / _N)
    d0 = z0 - m
    d1 = z1 - m
    v = (jnp.sum(d0 * d0, 0, keepdims=True)
         + jnp.sum(d1 * d1, 0, keepdims=True)) * (0.5 / _N)
    scale = lax.rsqrt(v + 1e-5) * g[...]
    r0 = jnp.maximum(d0 * scale + bt[...], 0.0)
    r1 = jnp.maximum(d1 * scale + bt[...], 0.0)
    o[pl.ds(0, _N), :] = jnp.maximum(
        jnp.dot(r0, w2t[...], preferred_element_type=jnp.float32) + b2[...], 0.0)
    o[pl.ds(_N, _N), :] = jnp.maximum(
        jnp.dot(r1, w2t[...], preferred_element_type=jnp.float32) + b2[...], 0.0)


_layer = pl.pallas_call(
    _layer_body,
    out_shape=jax.ShapeDtypeStruct((_R * _N, _HID), jnp.float32),
)


def _fin_body(hf, wft, bf, o):
    hm = (hf[pl.ds(0, _N), :] + hf[pl.ds(_N, _N), :]) * 0.5
    o[...] = jnp.dot(hm, wft[...], preferred_element_type=jnp.float32) + bf[...]


_fin = pl.pallas_call(
    _fin_body,
    out_shape=jax.ShapeDtypeStruct((_N, 40), jnp.float32),
)


def kernel(x_author, edge_index, W0, b0, W1_0, b1_0, g_0, bt_0, W2_0, b2_0,
           eps_0, W1_1, b1_1, g_1, bt_1, W2_1, b2_1, eps_1, Wf, bf):
    mask = jax.random.bernoulli(
        jax.random.key(42), 1.0 - _P, (_R, _N)).astype(jnp.float32)
    mask = mask * (1.0 / (1.0 - _P))
    m0 = mask[0][:, None]
    m1 = mask[1][:, None]
    eidx = jnp.stack([edge_index[0].reshape(_NW, _NCH, _K),
                      edge_index[1].reshape(_NW, _NCH, _K)], axis=2)

    hf = _proj(x_author, W0.T, b0.reshape(1, -1), m0, m1)

    w1ts = jnp.stack([W1_0.T, W1_1.T])
    b1s = jnp.stack([b1_0, b1_1]).reshape(_R, 1, _HID)
    gs = jnp.stack([g_0, g_1]).reshape(_R, 1, _HID)
    bts = jnp.stack([bt_0, bt_1]).reshape(_R, 1, _HID)
    w2ts = jnp.stack([W2_0.T, W2_1.T])
    b2s = jnp.stack([b2_0, b2_1]).reshape(_R, 1, _HID)
    epss = jnp.stack([eps_0, eps_1]).reshape(_R, 1, 1)

    nlayers = 2 + lax.optimization_barrier(jnp.zeros((), jnp.int32))
    stacked = (w1ts, b1s, gs, bts, w2ts, b2s, epss)

    def cond(carry):
        return carry[0] < nlayers

    def step(carry):
        i, hfc = carry
        w1t, b1r, gr, btr, w2t, b2r, epsr = (
            lax.dynamic_index_in_dim(w, i, keepdims=False) for w in stacked)
        agg = _get_sc_aggregate()(hfc, eidx)
        hfc = _layer(hfc, agg, w1t, b1r, gr, btr, w2t, b2r, epsr)
        return (i + 1, hfc)

    _, hf = lax.while_loop(cond, step, (jnp.int32(0), hf))

    return _fin(hf, Wf.T, bf.reshape(1, -1))
